# Initial kernel scaffold; baseline (speedup 1.0000x reference)
import jax
import jax.numpy as jnp
from jax.experimental import pallas as pl


def kernel(x, edge_weight, node_weight, node_bias):
    raise NotImplementedError("write your pallas kernel here")


# speedup vs baseline: 8.2593x; 8.2593x over previous
import functools

import jax
import jax.numpy as jnp
from jax import lax
from jax.experimental import pallas as pl
from jax.experimental.pallas import tpu as pltpu
from jax.experimental.pallas import tpu_sc as plsc

ROWS = 256
OPAD = 16
CHUNK = 128


def _topk_body(n, k, scale_ref, x_rows_ref, x_full_ref, w_ref, bias_ref,
               bond_ref, gidx_ref, sim_ref):
    b = pl.program_id(0)
    i = pl.program_id(1)
    w = w_ref[...]
    bias = bias_ref[...]

    def feat(v):
        t = lax.dot_general(v, w, (((1,), (1,)), ((), ())),
                            preferred_element_type=jnp.float32) + bias
        return jnp.where(t > 0, t, 0.01 * t)

    xt_rows = feat(x_rows_ref[0])
    xt_full = feat(x_full_ref[0])
    sim = lax.dot_general(xt_rows, xt_full, (((1,), (1,)), ((), ())),
                          preferred_element_type=jnp.float32)
    sim = sim * scale_ref[...]
    rows = i * ROWS + lax.broadcasted_iota(jnp.int32, (ROWS, n), 0)
    cols = lax.broadcasted_iota(jnp.int32, (ROWS, n), 1)
    sim_ref[...] = jnp.where(cols == rows, -jnp.inf, sim)

    kcols = lax.broadcasted_iota(jnp.int32, (ROWS, k), 1)

    def step(j, carry):
        vals, idxs = carry
        s = sim_ref[...]
        m = jnp.max(s, axis=1, keepdims=True)
        am = jnp.min(jnp.where(s == m, cols, n), axis=1,
                     keepdims=True)
        sim_ref[...] = jnp.where(cols == am, -jnp.inf, s)
        vals = jnp.where(kcols == j, m, vals)
        idxs = jnp.where(kcols == j, am, idxs)
        return vals, idxs

    vals0 = jnp.zeros((ROWS, k), jnp.float32)
    idxs0 = jnp.zeros((ROWS, k), jnp.int32)
    vals, idxs = lax.fori_loop(0, k, step, (vals0, idxs0))
    bond_ref[0] = 1.0 / (1.0 + jnp.exp(-vals))
    gidx_ref[0] = idxs + b * n


def _topk_call(x, w_pad, bias_pad, scale, k):
    bsz, n, d = x.shape
    grid = (bsz, n // ROWS)
    return pl.pallas_call(
        functools.partial(_topk_body, n, k),
        grid=grid,
        in_specs=[
            pl.BlockSpec((1, 1), lambda b, i: (0, 0)),
            pl.BlockSpec((1, ROWS, d), lambda b, i: (b, i, 0)),
            pl.BlockSpec((1, n, d), lambda b, i: (b, 0, 0)),
            pl.BlockSpec((OPAD, d), lambda b, i: (0, 0)),
            pl.BlockSpec((1, OPAD), lambda b, i: (0, 0)),
        ],
        out_specs=[
            pl.BlockSpec((1, ROWS, k), lambda b, i: (b, i, 0)),
            pl.BlockSpec((1, ROWS, k), lambda b, i: (b, i, 0)),
        ],
        out_shape=[
            jax.ShapeDtypeStruct((bsz, n, k), jnp.float32),
            jax.ShapeDtypeStruct((bsz, n, k), jnp.int32),
        ],
        scratch_shapes=[pltpu.VMEM((ROWS, n), jnp.float32)],
    )(scale, x, x, w_pad, bias_pad)


def _gather_call(table, flat_idx):
    total, d = table.shape[0], table.shape[1]
    g = flat_idx.shape[0]
    info = plsc.get_sparse_core_info()
    nw = info.num_cores * info.num_subcores
    per_w = g // nw
    mesh = plsc.VectorSubcoreMesh(core_axis_name="c", subcore_axis_name="s")

    @functools.partial(
        pl.kernel,
        out_type=jax.ShapeDtypeStruct((g, d), jnp.float32),
        mesh=mesh,
        scratch_types=[
            pltpu.VMEM((CHUNK,), jnp.int32),
            pltpu.VMEM((CHUNK, d), jnp.float32),
            pltpu.SemaphoreType.DMA,
        ],
    )
    def gather_k(table_hbm, idx_hbm, out_hbm, idx_v, rows_v, sem):
        wid = lax.axis_index("s") * info.num_cores + lax.axis_index("c")
        base = wid * per_w

        def body(j, carry):
            off = base + j * CHUNK
            pltpu.sync_copy(idx_hbm.at[pl.ds(off, CHUNK)], idx_v)
            pltpu.async_copy(table_hbm.at[idx_v], rows_v, sem).wait()
            pltpu.sync_copy(rows_v, out_hbm.at[pl.ds(off, CHUNK)])
            return carry

        lax.fori_loop(0, per_w // CHUNK, body, 0)

    return gather_k(table, flat_idx)


def kernel(x, edge_weight, node_weight, node_bias):
    bsz, n, d = x.shape
    k = 32
    o = node_weight.shape[0]
    w_pad = jnp.zeros((OPAD, d), jnp.float32).at[:o].set(node_weight)
    bias_pad = jnp.zeros((1, OPAD), jnp.float32).at[0, :o].set(node_bias)
    scale = jnp.exp(edge_weight).reshape(1, 1).astype(jnp.float32)

    bond, gidx = _topk_call(x, w_pad, bias_pad, scale, k)

    table = x.reshape(bsz * n, d)
    rows = _gather_call(table, gidx.reshape(bsz * n * k))
    node_neighbor = rows.reshape(bsz, n, k, d)
    bond_neighbor = bond.reshape(bsz, n, k, 1)
    return node_neighbor, bond_neighbor


# speedup vs baseline: 8.4085x; 1.0181x over previous
import functools

import jax
import jax.numpy as jnp
from jax import lax
from jax.experimental import pallas as pl
from jax.experimental.pallas import tpu as pltpu
from jax.experimental.pallas import tpu_sc as plsc

ROWS = 256
OPAD = 16
CHUNK = 128


def _topk_body(n, k, scale_ref, x_rows_ref, x_full_ref, w_ref, bias_ref,
               bond_ref, gidx_ref, sim_ref):
    b = pl.program_id(0)
    i = pl.program_id(1)
    w = w_ref[...]
    bias = bias_ref[...]

    def feat(v):
        t = lax.dot_general(v, w, (((1,), (1,)), ((), ())),
                            preferred_element_type=jnp.float32) + bias
        return jnp.where(t > 0, t, 0.01 * t)

    xt_rows = feat(x_rows_ref[0])
    xt_full = feat(x_full_ref[0])
    sim = lax.dot_general(xt_rows, xt_full, (((1,), (1,)), ((), ())),
                          preferred_element_type=jnp.float32)
    sim = sim * scale_ref[...]
    rows = i * ROWS + lax.broadcasted_iota(jnp.int32, (ROWS, n), 0)
    cols = lax.broadcasted_iota(jnp.int32, (ROWS, n), 1)
    sim_ref[...] = jnp.where(cols == rows, -jnp.inf, sim)

    kcols = lax.broadcasted_iota(jnp.int32, (ROWS, k), 1)

    def step(j, carry):
        vals, idxs = carry
        s = sim_ref[...]
        m = jnp.max(s, axis=1, keepdims=True)
        am = jnp.min(jnp.where(s == m, cols, n), axis=1,
                     keepdims=True)
        sim_ref[...] = jnp.where(cols == am, -jnp.inf, s)
        vals = jnp.where(kcols == j, m, vals)
        idxs = jnp.where(kcols == j, am, idxs)
        return vals, idxs

    vals0 = jnp.zeros((ROWS, k), jnp.float32)
    idxs0 = jnp.zeros((ROWS, k), jnp.int32)
    vals, idxs = lax.fori_loop(0, k, step, (vals0, idxs0))
    bond_ref[0] = 1.0 / (1.0 + jnp.exp(-vals))
    gidx_ref[0] = idxs + b * n


def _topk_call(x, w_pad, bias_pad, scale, k):
    bsz, n, d = x.shape
    grid = (bsz, n // ROWS)
    return pl.pallas_call(
        functools.partial(_topk_body, n, k),
        grid=grid,
        in_specs=[
            pl.BlockSpec((1, 1), lambda b, i: (0, 0)),
            pl.BlockSpec((1, ROWS, d), lambda b, i: (b, i, 0)),
            pl.BlockSpec((1, n, d), lambda b, i: (b, 0, 0)),
            pl.BlockSpec((OPAD, d), lambda b, i: (0, 0)),
            pl.BlockSpec((1, OPAD), lambda b, i: (0, 0)),
        ],
        out_specs=[
            pl.BlockSpec((1, ROWS, k), lambda b, i: (b, i, 0)),
            pl.BlockSpec((1, ROWS, k), lambda b, i: (b, i, 0)),
        ],
        out_shape=[
            jax.ShapeDtypeStruct((bsz, n, k), jnp.float32),
            jax.ShapeDtypeStruct((bsz, n, k), jnp.int32),
        ],
        scratch_shapes=[pltpu.VMEM((ROWS, n), jnp.float32)],
    )(scale, x, x, w_pad, bias_pad)


def _gather_call(table, flat_idx):
    total, d = table.shape[0], table.shape[1]
    g = flat_idx.shape[0]
    info = plsc.get_sparse_core_info()
    nw = info.num_cores * info.num_subcores
    per_w = g // nw
    mesh = plsc.VectorSubcoreMesh(core_axis_name="c", subcore_axis_name="s")

    nchunks = per_w // CHUNK

    @functools.partial(
        pl.kernel,
        out_type=jax.ShapeDtypeStruct((g, d), jnp.float32),
        mesh=mesh,
        scratch_types=[
            pltpu.VMEM((2, CHUNK), jnp.int32),
            pltpu.VMEM((2, CHUNK, d), jnp.float32),
            pltpu.SemaphoreType.DMA((2,)),
            pltpu.SemaphoreType.DMA((2,)),
            pltpu.SemaphoreType.DMA((2,)),
        ],
    )
    def gather_k(table_hbm, idx_hbm, out_hbm, idx_v, rows_v, isem, gsem,
                 ssem):
        wid = lax.axis_index("s") * info.num_cores + lax.axis_index("c")
        base = wid * per_w

        def idx_cp(j, s):
            return pltpu.make_async_copy(
                idx_hbm.at[pl.ds(base + j * CHUNK, CHUNK)], idx_v.at[s],
                isem.at[s])

        def gather_cp(s):
            return pltpu.make_async_copy(table_hbm.at[idx_v.at[s]],
                                         rows_v.at[s], gsem.at[s])

        def scatter_cp(j, s):
            return pltpu.make_async_copy(
                rows_v.at[s], out_hbm.at[pl.ds(base + j * CHUNK, CHUNK)],
                ssem.at[s])

        idx_cp(0, 0).start()

        def body(j, carry):
            s = j % 2
            idx_cp(j, s).wait()

            @pl.when(j >= 2)
            def _():
                scatter_cp(j - 2, s).wait()

            gather_cp(s).start()

            @pl.when(j >= 1)
            def _():
                gather_cp(1 - s).wait()
                scatter_cp(j - 1, 1 - s).start()

            @pl.when(j + 1 < nchunks)
            def _():
                idx_cp(j + 1, 1 - s).start()

            return carry

        lax.fori_loop(0, nchunks, body, 0)
        last = nchunks - 1
        s = last % 2
        gather_cp(s).wait()
        scatter_cp(last, s).start()
        scatter_cp(last - 1, 1 - s).wait()
        scatter_cp(last, s).wait()

    return gather_k(table, flat_idx)


def kernel(x, edge_weight, node_weight, node_bias):
    bsz, n, d = x.shape
    k = 32
    o = node_weight.shape[0]
    w_pad = jnp.zeros((OPAD, d), jnp.float32).at[:o].set(node_weight)
    bias_pad = jnp.zeros((1, OPAD), jnp.float32).at[0, :o].set(node_bias)
    scale = jnp.exp(edge_weight).reshape(1, 1).astype(jnp.float32)

    bond, gidx = _topk_call(x, w_pad, bias_pad, scale, k)

    table = x.reshape(bsz * n, d)
    rows = _gather_call(table, gidx.reshape(bsz * n * k))
    node_neighbor = rows.reshape(bsz, n, k, d)
    bond_neighbor = bond.reshape(bsz, n, k, 1)
    return node_neighbor, bond_neighbor


# speedup vs baseline: 10.1291x; 1.2046x over previous
import functools

import jax
import jax.numpy as jnp
from jax import lax
from jax.experimental import pallas as pl
from jax.experimental.pallas import tpu as pltpu
from jax.experimental.pallas import tpu_sc as plsc

ROWS = 256
OPAD = 16
CHUNK = 128


def _topk_body(n, k, scale_ref, x_rows_ref, x_full_ref, w_ref, bias_ref,
               bond_ref, gidx_ref, stack_ref, p2_ref):
    b = pl.program_id(0)
    i = pl.program_id(1)
    q = n // 4
    shq = q.bit_length() - 1
    w = w_ref[...]
    bias = bias_ref[...]

    def feat(v):
        t = lax.dot_general(v, w, (((1,), (1,)), ((), ())),
                            preferred_element_type=jnp.float32) + bias
        return jnp.where(t > 0, t, 0.01 * t)

    xt_rows = feat(x_rows_ref[0])
    xt_full = feat(x_full_ref[0])
    sim = lax.dot_general(xt_rows, xt_full, (((1,), (1,)), ((), ())),
                          preferred_element_type=jnp.float32)
    sim = sim * scale_ref[...]
    rows = i * ROWS + lax.broadcasted_iota(jnp.int32, (ROWS, n), 0)
    cols = lax.broadcasted_iota(jnp.int32, (ROWS, n), 1)
    sim = jnp.where(cols == rows, -jnp.inf, sim)

    parts = [lax.slice(sim, (0, m * q), (ROWS, (m + 1) * q))
             for m in range(4)]
    ids = [jnp.full((ROWS, q), m, jnp.int32) for m in range(4)]

    def cswap(a, ia, c, ic):
        sw = (c > a) | ((c == a) & (ic < ia))
        return (jnp.where(sw, c, a), jnp.where(sw, ic, ia),
                jnp.where(sw, a, c), jnp.where(sw, ia, ic))

    for x_, y_ in ((0, 1), (2, 3), (0, 2), (1, 3), (1, 2)):
        parts[x_], ids[x_], parts[y_], ids[y_] = cswap(
            parts[x_], ids[x_], parts[y_], ids[y_])
    for m in range(4):
        stack_ref[:, m * q:(m + 1) * q] = parts[m]
    p2_ref[...] = (ids[0] | (ids[1] << 2) | (ids[2] << 4) | (ids[3] << 6))

    lane = lax.broadcasted_iota(jnp.int32, (ROWS, q), 1)
    kcols = lax.broadcasted_iota(jnp.int32, (ROWS, k), 1)

    def step(j, carry):
        vals, idxs = carry
        a0 = stack_ref[:, :q]
        p2v = p2_ref[...]
        m = jnp.max(a0, axis=1, keepdims=True)
        oc = ((p2v & 3) << shq) | lane
        eq = a0 == m
        outcol = jnp.min(jnp.where(eq, oc, n), axis=1, keepdims=True)
        eqam = eq & (oc == outcol)
        a1 = stack_ref[:, q:2 * q]
        a2 = stack_ref[:, 2 * q:3 * q]
        a3 = stack_ref[:, 3 * q:]
        stack_ref[:, :q] = jnp.where(eqam, a1, a0)
        stack_ref[:, q:2 * q] = jnp.where(eqam, a2, a1)
        stack_ref[:, 2 * q:3 * q] = jnp.where(eqam, a3, a2)
        stack_ref[:, 3 * q:] = jnp.where(eqam, -jnp.inf, a3)
        p2_ref[...] = jnp.where(eqam, p2v >> 2, p2v)
        vals = jnp.where(kcols == j, m, vals)
        idxs = jnp.where(kcols == j, outcol, idxs)
        return vals, idxs

    vals0 = jnp.zeros((ROWS, k), jnp.float32)
    idxs0 = jnp.zeros((ROWS, k), jnp.int32)
    vals, idxs = lax.fori_loop(0, k, step, (vals0, idxs0))
    bond_ref[0] = 1.0 / (1.0 + jnp.exp(-vals))
    gidx_ref[0] = idxs + b * n


def _topk_call(x, w_pad, bias_pad, scale, k):
    bsz, n, d = x.shape
    grid = (bsz, n // ROWS)
    return pl.pallas_call(
        functools.partial(_topk_body, n, k),
        grid=grid,
        in_specs=[
            pl.BlockSpec((1, 1), lambda b, i: (0, 0)),
            pl.BlockSpec((1, ROWS, d), lambda b, i: (b, i, 0)),
            pl.BlockSpec((1, n, d), lambda b, i: (b, 0, 0)),
            pl.BlockSpec((OPAD, d), lambda b, i: (0, 0)),
            pl.BlockSpec((1, OPAD), lambda b, i: (0, 0)),
        ],
        out_specs=[
            pl.BlockSpec((1, ROWS, k), lambda b, i: (b, i, 0)),
            pl.BlockSpec((1, ROWS, k), lambda b, i: (b, i, 0)),
        ],
        out_shape=[
            jax.ShapeDtypeStruct((bsz, n, k), jnp.float32),
            jax.ShapeDtypeStruct((bsz, n, k), jnp.int32),
        ],
        scratch_shapes=[pltpu.VMEM((ROWS, n), jnp.float32),
                        pltpu.VMEM((ROWS, n // 4), jnp.int32)],
    )(scale, x, x, w_pad, bias_pad)


def _gather_call(table, flat_idx):
    total, d = table.shape[0], table.shape[1]
    g = flat_idx.shape[0]
    info = plsc.get_sparse_core_info()
    nw = info.num_cores * info.num_subcores
    per_w = g // nw
    mesh = plsc.VectorSubcoreMesh(core_axis_name="c", subcore_axis_name="s")

    nchunks = per_w // CHUNK

    @functools.partial(
        pl.kernel,
        out_type=jax.ShapeDtypeStruct((g, d), jnp.float32),
        mesh=mesh,
        scratch_types=[
            pltpu.VMEM((2, CHUNK), jnp.int32),
            pltpu.VMEM((2, CHUNK, d), jnp.float32),
            pltpu.SemaphoreType.DMA((2,)),
            pltpu.SemaphoreType.DMA((2,)),
            pltpu.SemaphoreType.DMA((2,)),
        ],
    )
    def gather_k(table_hbm, idx_hbm, out_hbm, idx_v, rows_v, isem, gsem,
                 ssem):
        wid = lax.axis_index("s") * info.num_cores + lax.axis_index("c")
        base = wid * per_w

        def idx_cp(j, s):
            return pltpu.make_async_copy(
                idx_hbm.at[pl.ds(base + j * CHUNK, CHUNK)], idx_v.at[s],
                isem.at[s])

        def gather_cp(s):
            return pltpu.make_async_copy(table_hbm.at[idx_v.at[s]],
                                         rows_v.at[s], gsem.at[s])

        def scatter_cp(j, s):
            return pltpu.make_async_copy(
                rows_v.at[s], out_hbm.at[pl.ds(base + j * CHUNK, CHUNK)],
                ssem.at[s])

        idx_cp(0, 0).start()

        def body(j, carry):
            s = j % 2
            idx_cp(j, s).wait()

            @pl.when(j >= 2)
            def _():
                scatter_cp(j - 2, s).wait()

            gather_cp(s).start()

            @pl.when(j >= 1)
            def _():
                gather_cp(1 - s).wait()
                scatter_cp(j - 1, 1 - s).start()

            @pl.when(j + 1 < nchunks)
            def _():
                idx_cp(j + 1, 1 - s).start()

            return carry

        lax.fori_loop(0, nchunks, body, 0)
        last = nchunks - 1
        s = last % 2
        gather_cp(s).wait()
        scatter_cp(last, s).start()
        scatter_cp(last - 1, 1 - s).wait()
        scatter_cp(last, s).wait()

    return gather_k(table, flat_idx)


def kernel(x, edge_weight, node_weight, node_bias):
    bsz, n, d = x.shape
    k = 32
    o = node_weight.shape[0]
    w_pad = jnp.zeros((OPAD, d), jnp.float32).at[:o].set(node_weight)
    bias_pad = jnp.zeros((1, OPAD), jnp.float32).at[0, :o].set(node_bias)
    scale = jnp.exp(edge_weight).reshape(1, 1).astype(jnp.float32)

    bond, gidx = _topk_call(x, w_pad, bias_pad, scale, k)

    table = x.reshape(bsz * n, d)
    rows = _gather_call(table, gidx.reshape(bsz * n * k))
    node_neighbor = rows.reshape(bsz, n, k, d)
    bond_neighbor = bond.reshape(bsz, n, k, 1)
    return node_neighbor, bond_neighbor


# speedup vs baseline: 12.4379x; 1.2279x over previous
import functools

import jax
import jax.numpy as jnp
from jax import lax
from jax.experimental import pallas as pl
from jax.experimental.pallas import tpu as pltpu
from jax.experimental.pallas import tpu_sc as plsc

ROWS = 256
OPAD = 16
CHUNK = 128


PARTS = 8
IDB = 3
SORT_NET = {
    4: ((0, 1), (2, 3), (0, 2), (1, 3), (1, 2)),
    8: ((0, 1), (2, 3), (4, 5), (6, 7),
        (0, 2), (1, 3), (4, 6), (5, 7),
        (1, 2), (5, 6),
        (0, 4), (1, 5), (2, 6), (3, 7),
        (2, 4), (3, 5),
        (1, 2), (3, 4), (5, 6)),
}


def _topk_body(n, k, scale_ref, x_rows_ref, x_full_ref, w_ref, bias_ref,
               bond_ref, gidx_ref, stack_ref, p2_ref):
    b = pl.program_id(0)
    i = pl.program_id(1)
    q = n // PARTS
    shq = q.bit_length() - 1
    w = w_ref[...]
    bias = bias_ref[...]

    def feat(v):
        t = lax.dot_general(v, w, (((1,), (1,)), ((), ())),
                            preferred_element_type=jnp.float32) + bias
        return jnp.where(t > 0, t, 0.01 * t)

    xt_rows = feat(x_rows_ref[0])
    xt_full = feat(x_full_ref[0])
    sim = lax.dot_general(xt_rows, xt_full, (((1,), (1,)), ((), ())),
                          preferred_element_type=jnp.float32)
    sim = sim * scale_ref[...]
    rows = i * ROWS + lax.broadcasted_iota(jnp.int32, (ROWS, n), 0)
    cols = lax.broadcasted_iota(jnp.int32, (ROWS, n), 1)
    sim = jnp.where(cols == rows, -jnp.inf, sim)

    parts = [lax.slice(sim, (0, m * q), (ROWS, (m + 1) * q))
             for m in range(PARTS)]
    ids = [jnp.full((ROWS, q), m, jnp.int32) for m in range(PARTS)]

    def cswap(a, ia, c, ic):
        sw = (c > a) | ((c == a) & (ic < ia))
        return (jnp.where(sw, c, a), jnp.where(sw, ic, ia),
                jnp.where(sw, a, c), jnp.where(sw, ia, ic))

    for x_, y_ in SORT_NET[PARTS]:
        parts[x_], ids[x_], parts[y_], ids[y_] = cswap(
            parts[x_], ids[x_], parts[y_], ids[y_])
    for m in range(PARTS):
        stack_ref[:, m * q:(m + 1) * q] = parts[m]
    packed = ids[0]
    for m in range(1, PARTS):
        packed = packed | (ids[m] << (IDB * m))
    p2_ref[...] = packed

    lane = lax.broadcasted_iota(jnp.int32, (ROWS, q), 1)
    kcols = lax.broadcasted_iota(jnp.int32, (ROWS, k), 1)

    def step(j, carry):
        vals, idxs = carry
        lv = [stack_ref[:, m * q:(m + 1) * q] for m in range(PARTS)]
        p2v = p2_ref[...]
        m = jnp.max(lv[0], axis=1, keepdims=True)
        oc = ((p2v & (PARTS - 1)) << shq) | lane
        eq = lv[0] == m
        outcol = jnp.min(jnp.where(eq, oc, n), axis=1, keepdims=True)
        eqam = eq & (oc == outcol)
        for d in range(PARTS - 1):
            stack_ref[:, d * q:(d + 1) * q] = jnp.where(
                eqam, lv[d + 1], lv[d])
        stack_ref[:, (PARTS - 1) * q:] = jnp.where(
            eqam, -jnp.inf, lv[PARTS - 1])
        p2_ref[...] = jnp.where(eqam, p2v >> IDB, p2v)
        vals = jnp.where(kcols == j, m, vals)
        idxs = jnp.where(kcols == j, outcol, idxs)
        return vals, idxs

    vals0 = jnp.zeros((ROWS, k), jnp.float32)
    idxs0 = jnp.zeros((ROWS, k), jnp.int32)
    vals, idxs = lax.fori_loop(0, k, step, (vals0, idxs0))
    bond_ref[0] = 1.0 / (1.0 + jnp.exp(-vals))
    gidx_ref[0] = idxs + b * n


def _topk_call(x, w_pad, bias_pad, scale, k):
    bsz, n, d = x.shape
    grid = (bsz, n // ROWS)
    return pl.pallas_call(
        functools.partial(_topk_body, n, k),
        grid=grid,
        in_specs=[
            pl.BlockSpec((1, 1), lambda b, i: (0, 0)),
            pl.BlockSpec((1, ROWS, d), lambda b, i: (b, i, 0)),
            pl.BlockSpec((1, n, d), lambda b, i: (b, 0, 0)),
            pl.BlockSpec((OPAD, d), lambda b, i: (0, 0)),
            pl.BlockSpec((1, OPAD), lambda b, i: (0, 0)),
        ],
        out_specs=[
            pl.BlockSpec((1, ROWS, k), lambda b, i: (b, i, 0)),
            pl.BlockSpec((1, ROWS, k), lambda b, i: (b, i, 0)),
        ],
        out_shape=[
            jax.ShapeDtypeStruct((bsz, n, k), jnp.float32),
            jax.ShapeDtypeStruct((bsz, n, k), jnp.int32),
        ],
        scratch_shapes=[pltpu.VMEM((ROWS, n), jnp.float32),
                        pltpu.VMEM((ROWS, n // PARTS), jnp.int32)],
    )(scale, x, x, w_pad, bias_pad)


def _gather_call(table, flat_idx):
    total, d = table.shape[0], table.shape[1]
    g = flat_idx.shape[0]
    info = plsc.get_sparse_core_info()
    nw = info.num_cores * info.num_subcores
    per_w = g // nw
    mesh = plsc.VectorSubcoreMesh(core_axis_name="c", subcore_axis_name="s")

    nchunks = per_w // CHUNK

    @functools.partial(
        pl.kernel,
        out_type=jax.ShapeDtypeStruct((g, d), jnp.float32),
        mesh=mesh,
        scratch_types=[
            pltpu.VMEM((2, CHUNK), jnp.int32),
            pltpu.VMEM((2, CHUNK, d), jnp.float32),
            pltpu.SemaphoreType.DMA((2,)),
            pltpu.SemaphoreType.DMA((2,)),
            pltpu.SemaphoreType.DMA((2,)),
        ],
    )
    def gather_k(table_hbm, idx_hbm, out_hbm, idx_v, rows_v, isem, gsem,
                 ssem):
        wid = lax.axis_index("s") * info.num_cores + lax.axis_index("c")
        base = wid * per_w

        def idx_cp(j, s):
            return pltpu.make_async_copy(
                idx_hbm.at[pl.ds(base + j * CHUNK, CHUNK)], idx_v.at[s],
                isem.at[s])

        def gather_cp(s):
            return pltpu.make_async_copy(table_hbm.at[idx_v.at[s]],
                                         rows_v.at[s], gsem.at[s])

        def scatter_cp(j, s):
            return pltpu.make_async_copy(
                rows_v.at[s], out_hbm.at[pl.ds(base + j * CHUNK, CHUNK)],
                ssem.at[s])

        idx_cp(0, 0).start()

        def body(j, carry):
            s = j % 2
            idx_cp(j, s).wait()

            @pl.when(j >= 2)
            def _():
                scatter_cp(j - 2, s).wait()

            gather_cp(s).start()

            @pl.when(j >= 1)
            def _():
                gather_cp(1 - s).wait()
                scatter_cp(j - 1, 1 - s).start()

            @pl.when(j + 1 < nchunks)
            def _():
                idx_cp(j + 1, 1 - s).start()

            return carry

        lax.fori_loop(0, nchunks, body, 0)
        last = nchunks - 1
        s = last % 2
        gather_cp(s).wait()
        scatter_cp(last, s).start()
        scatter_cp(last - 1, 1 - s).wait()
        scatter_cp(last, s).wait()

    return gather_k(table, flat_idx)


def kernel(x, edge_weight, node_weight, node_bias):
    bsz, n, d = x.shape
    k = 32
    o = node_weight.shape[0]
    w_pad = jnp.zeros((OPAD, d), jnp.float32).at[:o].set(node_weight)
    bias_pad = jnp.zeros((1, OPAD), jnp.float32).at[0, :o].set(node_bias)
    scale = jnp.exp(edge_weight).reshape(1, 1).astype(jnp.float32)

    bond, gidx = _topk_call(x, w_pad, bias_pad, scale, k)

    table = x.reshape(bsz * n, d)
    rows = _gather_call(table, gidx.reshape(bsz * n * k))
    node_neighbor = rows.reshape(bsz, n, k, d)
    bond_neighbor = bond.reshape(bsz, n, k, 1)
    return node_neighbor, bond_neighbor


# speedup vs baseline: 12.4568x; 1.0015x over previous
import functools

import jax
import jax.numpy as jnp
from jax import lax
from jax.experimental import pallas as pl
from jax.experimental.pallas import tpu as pltpu
from jax.experimental.pallas import tpu_sc as plsc

ROWS = 256
OPAD = 16
CHUNK = 128


PARTS = 8
IDB = 3
SORT_NET = {
    4: ((0, 1), (2, 3), (0, 2), (1, 3), (1, 2)),
    8: ((0, 1), (2, 3), (4, 5), (6, 7),
        (0, 2), (1, 3), (4, 6), (5, 7),
        (1, 2), (5, 6),
        (0, 4), (1, 5), (2, 6), (3, 7),
        (2, 4), (3, 5),
        (1, 2), (3, 4), (5, 6)),
}


def _topk_body(n, k, scale_ref, x_rows_ref, x_full_ref, w_ref, bias_ref,
               bond_ref, gidx_ref, stack_ref, p2_ref, oc_ref, d_ref):
    b = pl.program_id(0)
    i = pl.program_id(1)
    q = n // PARTS
    shq = q.bit_length() - 1
    w = w_ref[...]
    bias = bias_ref[...]

    def feat(v):
        t = lax.dot_general(v, w, (((1,), (1,)), ((), ())),
                            preferred_element_type=jnp.float32) + bias
        return jnp.where(t > 0, t, 0.01 * t)

    xt_rows = feat(x_rows_ref[0])
    xt_full = feat(x_full_ref[0])
    sim = lax.dot_general(xt_rows, xt_full, (((1,), (1,)), ((), ())),
                          preferred_element_type=jnp.float32)
    sim = sim * scale_ref[...]
    rows = i * ROWS + lax.broadcasted_iota(jnp.int32, (ROWS, n), 0)
    cols = lax.broadcasted_iota(jnp.int32, (ROWS, n), 1)
    sim = jnp.where(cols == rows, -jnp.inf, sim)

    parts = [lax.slice(sim, (0, m * q), (ROWS, (m + 1) * q))
             for m in range(PARTS)]
    ids = [jnp.full((ROWS, q), m, jnp.int32) for m in range(PARTS)]

    def cswap(a, ia, c, ic):
        sw = (c > a) | ((c == a) & (ic < ia))
        return (jnp.where(sw, c, a), jnp.where(sw, ic, ia),
                jnp.where(sw, a, c), jnp.where(sw, ia, ic))

    for x_, y_ in SORT_NET[PARTS]:
        parts[x_], ids[x_], parts[y_], ids[y_] = cswap(
            parts[x_], ids[x_], parts[y_], ids[y_])
    lane = lax.broadcasted_iota(jnp.int32, (ROWS, q), 1)
    for m in range(PARTS):
        stack_ref[:, m * q:(m + 1) * q] = parts[m]
    packed = ids[0]
    for m in range(1, PARTS):
        packed = packed | (ids[m] << (IDB * m))
    p2_ref[...] = packed
    oc_ref[...] = (ids[0] << shq) | lane
    d_ref[...] = jnp.zeros((ROWS, q), jnp.int32)

    kcols = lax.broadcasted_iota(jnp.int32, (ROWS, k), 1)

    def step(j, carry):
        vals, idxs = carry
        head = stack_ref[:, :q]
        ocv = oc_ref[...]
        m = jnp.max(head, axis=1, keepdims=True)
        eq = head == m
        outcol = jnp.min(jnp.where(eq, ocv, n), axis=1, keepdims=True)
        eqam = eq & (ocv == outcol)
        dn = d_ref[...] + 1
        nxt = jnp.full((ROWS, q), -jnp.inf)
        for lv in range(PARTS - 1, 0, -1):
            nxt = jnp.where(dn == lv, stack_ref[:, lv * q:(lv + 1) * q],
                            nxt)
        newmem = (p2_ref[...] >> (dn * IDB)) & (PARTS - 1)
        stack_ref[:, :q] = jnp.where(eqam, nxt, head)
        oc_ref[...] = jnp.where(eqam, (newmem << shq) | lane, ocv)
        d_ref[...] = jnp.where(eqam, dn, dn - 1)
        vals = jnp.where(kcols == j, m, vals)
        idxs = jnp.where(kcols == j, outcol, idxs)
        return vals, idxs

    vals0 = jnp.zeros((ROWS, k), jnp.float32)
    idxs0 = jnp.zeros((ROWS, k), jnp.int32)
    vals, idxs = lax.fori_loop(0, k, step, (vals0, idxs0))
    bond_ref[0] = 1.0 / (1.0 + jnp.exp(-vals))
    gidx_ref[0] = idxs + b * n


def _topk_call(x, w_pad, bias_pad, scale, k):
    bsz, n, d = x.shape
    grid = (bsz, n // ROWS)
    return pl.pallas_call(
        functools.partial(_topk_body, n, k),
        grid=grid,
        in_specs=[
            pl.BlockSpec((1, 1), lambda b, i: (0, 0)),
            pl.BlockSpec((1, ROWS, d), lambda b, i: (b, i, 0)),
            pl.BlockSpec((1, n, d), lambda b, i: (b, 0, 0)),
            pl.BlockSpec((OPAD, d), lambda b, i: (0, 0)),
            pl.BlockSpec((1, OPAD), lambda b, i: (0, 0)),
        ],
        out_specs=[
            pl.BlockSpec((1, ROWS, k), lambda b, i: (b, i, 0)),
            pl.BlockSpec((1, ROWS, k), lambda b, i: (b, i, 0)),
        ],
        out_shape=[
            jax.ShapeDtypeStruct((bsz, n, k), jnp.float32),
            jax.ShapeDtypeStruct((bsz, n, k), jnp.int32),
        ],
        scratch_shapes=[pltpu.VMEM((ROWS, n), jnp.float32),
                        pltpu.VMEM((ROWS, n // PARTS), jnp.int32),
                        pltpu.VMEM((ROWS, n // PARTS), jnp.int32),
                        pltpu.VMEM((ROWS, n // PARTS), jnp.int32)],
    )(scale, x, x, w_pad, bias_pad)


def _gather_call(table, flat_idx):
    total, d = table.shape[0], table.shape[1]
    g = flat_idx.shape[0]
    info = plsc.get_sparse_core_info()
    nw = info.num_cores * info.num_subcores
    per_w = g // nw
    mesh = plsc.VectorSubcoreMesh(core_axis_name="c", subcore_axis_name="s")

    nchunks = per_w // CHUNK

    @functools.partial(
        pl.kernel,
        out_type=jax.ShapeDtypeStruct((g, d), jnp.float32),
        mesh=mesh,
        scratch_types=[
            pltpu.VMEM((2, CHUNK), jnp.int32),
            pltpu.VMEM((2, CHUNK, d), jnp.float32),
            pltpu.SemaphoreType.DMA((2,)),
            pltpu.SemaphoreType.DMA((2,)),
            pltpu.SemaphoreType.DMA((2,)),
        ],
    )
    def gather_k(table_hbm, idx_hbm, out_hbm, idx_v, rows_v, isem, gsem,
                 ssem):
        wid = lax.axis_index("s") * info.num_cores + lax.axis_index("c")
        base = wid * per_w

        def idx_cp(j, s):
            return pltpu.make_async_copy(
                idx_hbm.at[pl.ds(base + j * CHUNK, CHUNK)], idx_v.at[s],
                isem.at[s])

        def gather_cp(s):
            return pltpu.make_async_copy(table_hbm.at[idx_v.at[s]],
                                         rows_v.at[s], gsem.at[s])

        def scatter_cp(j, s):
            return pltpu.make_async_copy(
                rows_v.at[s], out_hbm.at[pl.ds(base + j * CHUNK, CHUNK)],
                ssem.at[s])

        idx_cp(0, 0).start()

        def body(j, carry):
            s = j % 2
            idx_cp(j, s).wait()

            @pl.when(j >= 2)
            def _():
                scatter_cp(j - 2, s).wait()

            gather_cp(s).start()

            @pl.when(j >= 1)
            def _():
                gather_cp(1 - s).wait()
                scatter_cp(j - 1, 1 - s).start()

            @pl.when(j + 1 < nchunks)
            def _():
                idx_cp(j + 1, 1 - s).start()

            return carry

        lax.fori_loop(0, nchunks, body, 0)
        last = nchunks - 1
        s = last % 2
        gather_cp(s).wait()
        scatter_cp(last, s).start()
        scatter_cp(last - 1, 1 - s).wait()
        scatter_cp(last, s).wait()

    return gather_k(table, flat_idx)


def kernel(x, edge_weight, node_weight, node_bias):
    bsz, n, d = x.shape
    k = 32
    o = node_weight.shape[0]
    w_pad = jnp.zeros((OPAD, d), jnp.float32).at[:o].set(node_weight)
    bias_pad = jnp.zeros((1, OPAD), jnp.float32).at[0, :o].set(node_bias)
    scale = jnp.exp(edge_weight).reshape(1, 1).astype(jnp.float32)

    bond, gidx = _topk_call(x, w_pad, bias_pad, scale, k)

    table = x.reshape(bsz * n, d)
    rows = _gather_call(table, gidx.reshape(bsz * n * k))
    node_neighbor = rows.reshape(bsz, n, k, d)
    bond_neighbor = bond.reshape(bsz, n, k, 1)
    return node_neighbor, bond_neighbor
